# baseline (speedup 1.0000x reference)
import jax
import jax.numpy as jnp
from jax.experimental import pallas as pl

EPS = 1e-6
H = 64
L = 4
K = 16
N = 1024
TN = 256


def _bdot(a, b):
    return jnp.dot(a.astype(jnp.bfloat16), b.astype(jnp.bfloat16),
                   preferred_element_type=jnp.float32)


def _vec_act_rows(v, k):
    kn = jnp.sqrt(k[0] * k[0] + k[1] * k[1] + k[2] * k[2]) + EPS
    dot = (v[0] * k[0] + v[1] * k[1] + v[2] * k[2]) / kn
    coef = jnp.minimum(dot, 0.0) / kn
    return [v[s] - coef * k[s] for s in range(3)]


def _stage1_body(x_ref, xt_ref, wint_ref, dint_ref, h_ref):
    xb = x_ref[0]
    xt = xt_ref[0]

    d2 = None
    for s in range(3):
        diff = xt[:, s:s + 1] - xb[s:s + 1, :]
        sq = diff * diff
        d2 = sq if d2 is None else d2 + sq

    iota = jax.lax.broadcasted_iota(jnp.int32, (TN, N), 1)

    base = pl.program_id(1) * TN
    rowid = jax.lax.broadcasted_iota(jnp.int32, (TN, N), 0) + base
    d2 = jnp.where(iota == rowid, jnp.inf, d2)

    n2x = xt[:, 0:1] ** 2 + xt[:, 1:2] ** 2 + xt[:, 2:3] ** 2
    nx = jnp.maximum(jnp.sqrt(n2x), 1e-12)
    xd = [xt[:, s:s + 1] / nx for s in range(3)]

    wint = wint_ref[...]
    dint = dint_ref[...]

    def edge(q):
        e = [q[s] - xt[:, s:s + 1] for s in range(3)]
        c = [xd[1] * q[2] - xd[2] * q[1],
             xd[2] * q[0] - xd[0] * q[2],
             xd[0] * q[1] - xd[1] * q[0]]
        v = []
        k = []
        for s in range(3):
            ys = jnp.concatenate([c[s], e[s], xt[:, s:s + 1]], axis=1)
            vs = _bdot(ys, wint)
            v.append(vs)
            k.append(_bdot(vs, dint))
        return _vec_act_rows(v, k)

    out = edge([xt[:, s:s + 1] for s in range(3)])
    h = list(out)
    for _ in range(K - 1):
        m = jnp.min(d2, axis=1, keepdims=True)
        oh = d2 <= m
        d2 = jnp.where(oh, jnp.inf, d2)
        q = [jnp.sum(jnp.where(oh, xb[s:s + 1, :], 0.0),
                     axis=1, keepdims=True) for s in range(3)]
        out = edge(q)
        for s in range(3):
            h[s] = h[s] + out[s]

    scale = 1.0 / K
    for s in range(3):
        h_ref[0, s] = h[s] * scale


def _stage2_body(h_ref, wlt_ref, dlt_ref, wgt_ref, dgt_ref, wot_ref,
                 xo_ref, xm_ref):
    yv = [h_ref[0, s] for s in range(3)]

    feats = []
    for i in range(L):
        t1 = [_bdot(yv[s], wlt_ref[i]) for s in range(3)]
        yv = _vec_act_rows(t1, [_bdot(t1[s], dlt_ref[i]) for s in range(3)])
        yg = [jnp.mean(yv[s], axis=0, keepdims=True) for s in range(3)]
        cat = [jnp.concatenate(
            [yv[s], jnp.broadcast_to(yg[s], (N, H))], axis=1)
            for s in range(3)]
        t2 = [_bdot(cat[s], wgt_ref[i]) for s in range(3)]
        yv = _vec_act_rows(t2, [_bdot(t2[s], dgt_ref[i]) for s in range(3)])
        feats.append(yv)

    wot = wot_ref[...]
    for s in range(3):
        xc = jnp.concatenate([feats[i][s] for i in range(L)], axis=1)
        xo = _bdot(xc, wot)
        xo_ref[0, s] = xo
        xm_ref[0, s] = jnp.mean(xo, axis=0, keepdims=True)


@jax.jit
def kernel(x, W_in, D_in, W_layers, D_layers, W_glayers, D_glayers, W_out):
    B = x.shape[0]
    xT = jnp.transpose(x, (0, 2, 1))
    h = pl.pallas_call(
        _stage1_body,
        grid=(B, N // TN),
        in_specs=[
            pl.BlockSpec((1, 3, N), lambda b, t: (b, 0, 0)),
            pl.BlockSpec((1, TN, 3), lambda b, t: (b, t, 0)),
            pl.BlockSpec((3, H), lambda b, t: (0, 0)),
            pl.BlockSpec((H, H), lambda b, t: (0, 0)),
        ],
        out_specs=pl.BlockSpec((1, 3, TN, H), lambda b, t: (b, 0, t, 0)),
        out_shape=jax.ShapeDtypeStruct((B, 3, N, H), jnp.float32),
    )(x, xT, W_in.T, D_in.T)

    xo, xm = pl.pallas_call(
        _stage2_body,
        grid=(B,),
        in_specs=[
            pl.BlockSpec((1, 3, N, H), lambda b: (b, 0, 0, 0)),
            pl.BlockSpec((L, H, H), lambda b: (0, 0, 0)),
            pl.BlockSpec((L, H, H), lambda b: (0, 0, 0)),
            pl.BlockSpec((L, 2 * H, H), lambda b: (0, 0, 0)),
            pl.BlockSpec((L, H, H), lambda b: (0, 0, 0)),
            pl.BlockSpec((L * H, H), lambda b: (0, 0)),
        ],
        out_specs=[
            pl.BlockSpec((1, 3, N, H), lambda b: (b, 0, 0, 0)),
            pl.BlockSpec((1, 3, 1, H), lambda b: (b, 0, 0, 0)),
        ],
        out_shape=[
            jax.ShapeDtypeStruct((B, 3, N, H), jnp.float32),
            jax.ShapeDtypeStruct((B, 3, 1, H), jnp.float32),
        ],
    )(h,
      jnp.transpose(W_layers, (0, 2, 1)),
      jnp.transpose(D_layers, (0, 2, 1)),
      jnp.transpose(W_glayers, (0, 2, 1)),
      jnp.transpose(D_glayers, (0, 2, 1)),
      W_out.T)

    xo_out = jnp.transpose(xo, (0, 3, 1, 2))
    mean_out = jnp.transpose(xm[:, :, 0, :], (0, 2, 1))
    return mean_out, xo_out


# speedup vs baseline: 1.0535x; 1.0535x over previous
import jax
import jax.numpy as jnp
from jax.experimental import pallas as pl

EPS = 1e-6
H = 64
L = 4
K = 16
N = 1024
TN = 512


def _bdot(a, b):
    return jnp.dot(a.astype(jnp.bfloat16), b.astype(jnp.bfloat16),
                   preferred_element_type=jnp.float32)


def _vec_act_rows(v, k):
    kn = jnp.sqrt(k[0] * k[0] + k[1] * k[1] + k[2] * k[2]) + EPS
    kdir = [k[s] / kn for s in range(3)]
    dot = v[0] * kdir[0] + v[1] * kdir[1] + v[2] * kdir[2]
    coef = jnp.minimum(dot, 0.0)
    return [v[s] - coef * kdir[s] for s in range(3)]


def _stage1_body(x_ref, xt_ref, wint_ref, dint_ref, h_ref):
    xb = x_ref[0]
    xt = xt_ref[0]

    d2 = None
    for s in range(3):
        diff = xt[:, s:s + 1] - xb[s:s + 1, :]
        sq = diff * diff
        d2 = sq if d2 is None else d2 + sq

    iota = jax.lax.broadcasted_iota(jnp.int32, (TN, N), 1)

    base = pl.program_id(1) * TN
    rowid = jax.lax.broadcasted_iota(jnp.int32, (TN, N), 0) + base
    d2 = jnp.where(iota == rowid, jnp.inf, d2)

    n2x = xt[:, 0:1] ** 2 + xt[:, 1:2] ** 2 + xt[:, 2:3] ** 2
    nx = jnp.maximum(jnp.sqrt(n2x), 1e-12)
    xd = [xt[:, s:s + 1] / nx for s in range(3)]

    wint = wint_ref[...]
    dint = dint_ref[...]

    def edge(q):
        e = [q[s] - xt[:, s:s + 1] for s in range(3)]
        c = [xd[1] * q[2] - xd[2] * q[1],
             xd[2] * q[0] - xd[0] * q[2],
             xd[0] * q[1] - xd[1] * q[0]]
        v = []
        k = []
        for s in range(3):
            ys = jnp.concatenate([c[s], e[s], xt[:, s:s + 1]], axis=1)
            vs = _bdot(ys, wint)
            v.append(vs)
            k.append(_bdot(vs, dint))
        return _vec_act_rows(v, k)

    out = edge([xt[:, s:s + 1] for s in range(3)])
    h = list(out)
    for _ in range(K - 1):
        m = jnp.min(d2, axis=1, keepdims=True)
        oh = d2 <= m
        d2 = jnp.where(oh, jnp.inf, d2)
        q = [jnp.sum(jnp.where(oh, xb[s:s + 1, :], 0.0),
                     axis=1, keepdims=True) for s in range(3)]
        out = edge(q)
        for s in range(3):
            h[s] = h[s] + out[s]

    scale = 1.0 / K
    for s in range(3):
        h_ref[0, s] = h[s] * scale


def _stage2_body(h_ref, wlt_ref, dlt_ref, wgt_ref, dgt_ref, wot_ref,
                 xo_ref, xm_ref):
    yv = [h_ref[0, s] for s in range(3)]

    feats = []
    for i in range(L):
        t1 = [_bdot(yv[s], wlt_ref[i]) for s in range(3)]
        yv = _vec_act_rows(t1, [_bdot(t1[s], dlt_ref[i]) for s in range(3)])
        yg = [jnp.mean(yv[s], axis=0, keepdims=True) for s in range(3)]
        cat = [jnp.concatenate(
            [yv[s], jnp.broadcast_to(yg[s], (N, H))], axis=1)
            for s in range(3)]
        t2 = [_bdot(cat[s], wgt_ref[i]) for s in range(3)]
        yv = _vec_act_rows(t2, [_bdot(t2[s], dgt_ref[i]) for s in range(3)])
        feats.append(yv)

    wot = wot_ref[...]
    for s in range(3):
        xc = jnp.concatenate([feats[i][s] for i in range(L)], axis=1)
        xo = _bdot(xc, wot)
        xo_ref[0, s] = xo
        xm_ref[0, s] = jnp.mean(xo, axis=0, keepdims=True)


@jax.jit
def kernel(x, W_in, D_in, W_layers, D_layers, W_glayers, D_glayers, W_out):
    B = x.shape[0]
    xT = jnp.transpose(x, (0, 2, 1))
    h = pl.pallas_call(
        _stage1_body,
        grid=(B, N // TN),
        in_specs=[
            pl.BlockSpec((1, 3, N), lambda b, t: (b, 0, 0)),
            pl.BlockSpec((1, TN, 3), lambda b, t: (b, t, 0)),
            pl.BlockSpec((3, H), lambda b, t: (0, 0)),
            pl.BlockSpec((H, H), lambda b, t: (0, 0)),
        ],
        out_specs=pl.BlockSpec((1, 3, TN, H), lambda b, t: (b, 0, t, 0)),
        out_shape=jax.ShapeDtypeStruct((B, 3, N, H), jnp.float32),
    )(x, xT, W_in.T, D_in.T)

    xo, xm = pl.pallas_call(
        _stage2_body,
        grid=(B,),
        in_specs=[
            pl.BlockSpec((1, 3, N, H), lambda b: (b, 0, 0, 0)),
            pl.BlockSpec((L, H, H), lambda b: (0, 0, 0)),
            pl.BlockSpec((L, H, H), lambda b: (0, 0, 0)),
            pl.BlockSpec((L, 2 * H, H), lambda b: (0, 0, 0)),
            pl.BlockSpec((L, H, H), lambda b: (0, 0, 0)),
            pl.BlockSpec((L * H, H), lambda b: (0, 0)),
        ],
        out_specs=[
            pl.BlockSpec((1, 3, N, H), lambda b: (b, 0, 0, 0)),
            pl.BlockSpec((1, 3, 1, H), lambda b: (b, 0, 0, 0)),
        ],
        out_shape=[
            jax.ShapeDtypeStruct((B, 3, N, H), jnp.float32),
            jax.ShapeDtypeStruct((B, 3, 1, H), jnp.float32),
        ],
    )(h,
      jnp.transpose(W_layers, (0, 2, 1)),
      jnp.transpose(D_layers, (0, 2, 1)),
      jnp.transpose(W_glayers, (0, 2, 1)),
      jnp.transpose(D_glayers, (0, 2, 1)),
      W_out.T)

    xo_out = jnp.transpose(xo, (0, 3, 1, 2))
    mean_out = jnp.transpose(xm[:, :, 0, :], (0, 2, 1))
    return mean_out, xo_out
